# baseline (speedup 1.0000x reference)
import functools
import jax
import jax.numpy as jnp
from jax import lax
from jax.experimental import pallas as pl
from jax.experimental.pallas import tpu as pltpu
from jax.experimental.pallas import tpu_sc as plsc

_T = 512

_NC = 2
_NS = 16
_NW = _NC * _NS


def _fill_scratch(cb_ref, cbm2_s, cb2_s):
    @pl.when(pl.program_id(0) == 0)
    def _():
        cb = cb_ref[0]
        cbm2_s[...] = (-2.0 * cb).astype(jnp.bfloat16)
        cb2_s[...] = jnp.sum(cb * cb, axis=1)[None, :]


def _argmin_dist(r, cbm2, cb2row, qoff):
    T = r.shape[0]
    K = cbm2.shape[0]
    mmn = lax.dot_general(
        r.astype(jnp.bfloat16), cbm2,
        (((1,), (1,)), ((), ())),
        preferred_element_type=jnp.float32,
    )
    r2 = jnp.sum(r * r, axis=1, keepdims=True)
    dist = (r2 + mmn) + cb2row
    m = jnp.min(dist, axis=1, keepdims=True)
    iota = lax.broadcasted_iota(jnp.int32, (1, K), 1).astype(jnp.float32)
    idxf = jnp.min(jnp.where(dist == m, iota, float(K)), axis=1)
    return idxf.astype(jnp.int32), (idxf + float(qoff)).astype(jnp.int32)


def _make_body_first(qoff):
    def body(x_ref, cb_ref, idx_ref, idxg_ref, cbm2_s, cb2_s):
        _fill_scratch(cb_ref, cbm2_s, cb2_s)
        il, ig = _argmin_dist(x_ref[...], cbm2_s[...], cb2_s[...], qoff)
        idx_ref[0, :] = il
        idxg_ref[0, :] = ig
    return body


def _make_body_mid(qoff):
    def body(rprev_ref, quant_ref, cb_ref, idx_ref, idxg_ref, r_ref, lp_ref,
             cbm2_s, cb2_s):
        _fill_scratch(cb_ref, cbm2_s, cb2_s)
        r = rprev_ref[...] - quant_ref[...]
        r_ref[...] = r
        lp_ref[0, 0, :] = jnp.sum(r * r, axis=0)
        il, ig = _argmin_dist(r, cbm2_s[...], cb2_s[...], qoff)
        idx_ref[0, :] = il
        idxg_ref[0, :] = ig
    return body


def _body_final(x_ref, rprev_ref, quant_ref, xq_ref, lp_ref):
    r = rprev_ref[...] - quant_ref[...]
    lp_ref[0, 0, :] = jnp.sum(r * r, axis=0)
    xq_ref[...] = x_ref[...] - r


def _tc_first(x, codebooks, q):
    N, D = x.shape
    Q, K, _ = codebooks.shape
    G = N // _T
    return pl.pallas_call(
        _make_body_first(q * K),
        grid=(G,),
        in_specs=[
            pl.BlockSpec((_T, D), lambda i: (i, 0)),
            pl.BlockSpec((1, K, D), lambda i: (q, 0, 0)),
        ],
        out_specs=[
            pl.BlockSpec((1, _T), lambda i: (0, i)),
            pl.BlockSpec((1, _T), lambda i: (0, i)),
        ],
        out_shape=[
            jax.ShapeDtypeStruct((1, N), jnp.int32),
            jax.ShapeDtypeStruct((1, N), jnp.int32),
        ],
        scratch_shapes=[
            pltpu.VMEM((K, D), jnp.bfloat16),
            pltpu.VMEM((1, K), jnp.float32),
        ],
    )(x, codebooks)


def _tc_mid(rprev, quant, codebooks, q):
    N, D = rprev.shape
    Q, K, _ = codebooks.shape
    G = N // _T
    return pl.pallas_call(
        _make_body_mid(q * K),
        grid=(G,),
        in_specs=[
            pl.BlockSpec((_T, D), lambda i: (i, 0)),
            pl.BlockSpec((_T, D), lambda i: (i, 0)),
            pl.BlockSpec((1, K, D), lambda i: (q, 0, 0)),
        ],
        out_specs=[
            pl.BlockSpec((1, _T), lambda i: (0, i)),
            pl.BlockSpec((1, _T), lambda i: (0, i)),
            pl.BlockSpec((_T, D), lambda i: (i, 0)),
            pl.BlockSpec((1, 1, D), lambda i: (i, 0, 0)),
        ],
        out_shape=[
            jax.ShapeDtypeStruct((1, N), jnp.int32),
            jax.ShapeDtypeStruct((1, N), jnp.int32),
            jax.ShapeDtypeStruct((N, D), jnp.float32),
            jax.ShapeDtypeStruct((G, 1, D), jnp.float32),
        ],
        scratch_shapes=[
            pltpu.VMEM((K, D), jnp.bfloat16),
            pltpu.VMEM((1, K), jnp.float32),
        ],
    )(rprev, quant, codebooks)


def _tc_final(x, rprev, quant):
    N, D = x.shape
    G = N // _T
    return pl.pallas_call(
        _body_final,
        grid=(G,),
        in_specs=[
            pl.BlockSpec((_T, D), lambda i: (i, 0)),
            pl.BlockSpec((_T, D), lambda i: (i, 0)),
            pl.BlockSpec((_T, D), lambda i: (i, 0)),
        ],
        out_specs=[
            pl.BlockSpec((_T, D), lambda i: (i, 0)),
            pl.BlockSpec((1, 1, D), lambda i: (i, 0, 0)),
        ],
        out_shape=[
            jax.ShapeDtypeStruct((N, D), jnp.float32),
            jax.ShapeDtypeStruct((G, 1, D), jnp.float32),
        ],
    )(x, rprev, quant)


def _sc_gather(flat_cb, idxg):
    KQ, D = flat_cb.shape
    N = idxg.shape[1]
    bpw = N // _NW
    nchunk = bpw // 128
    mesh = plsc.VectorSubcoreMesh(core_axis_name="c", subcore_axis_name="s")

    @functools.partial(
        pl.kernel, mesh=mesh,
        compiler_params=pltpu.CompilerParams(use_tc_tiling_on_sc=False),
        out_type=jax.ShapeDtypeStruct((N, D), jnp.float32),
        scratch_types=[
            pltpu.VMEM((bpw,), jnp.int32),
            pltpu.VMEM((bpw, D), jnp.float32),
            pltpu.SemaphoreType.DMA,
        ],
    )
    def k(cb_hbm, idx_hbm, out_hbm, idx_v, rows_v, sem):
        wid = lax.axis_index("s") * _NC + lax.axis_index("c")
        pltpu.sync_copy(idx_hbm.at[0, pl.ds(wid * bpw, bpw)], idx_v)
        copies = [
            pltpu.async_copy(
                cb_hbm.at[idx_v.at[pl.ds(j * 128, 128)]],
                rows_v.at[pl.ds(j * 128, 128)], sem)
            for j in range(nchunk)
        ]
        for c in copies:
            c.wait()
        pltpu.sync_copy(rows_v, out_hbm.at[pl.ds(wid * bpw, bpw)])

    return k(flat_cb, idxg)


def kernel(x, codebooks):
    B, S, D = x.shape
    Q, K, _ = codebooks.shape
    N = B * S
    xf = x.reshape(N, D)
    flat_cb = codebooks.reshape(Q * K, D)

    idx0, ig0 = _tc_first(xf, codebooks, 0)
    q0 = _sc_gather(flat_cb, ig0)
    idx1, ig1, r1, lp0 = _tc_mid(xf, q0, codebooks, 1)
    q1 = _sc_gather(flat_cb, ig1)
    idx2, ig2, r2, lp1 = _tc_mid(r1, q1, codebooks, 2)
    q2 = _sc_gather(flat_cb, ig2)
    idx3, ig3, r3, lp2 = _tc_mid(r2, q2, codebooks, 3)
    q3 = _sc_gather(flat_cb, ig3)
    xq_f, lp3 = _tc_final(xf, r3, q3)

    xq = xq_f.reshape(B, S, D)
    indices = jnp.stack(
        [idx0[0], idx1[0], idx2[0], idx3[0]], axis=-1).reshape(B, S, Q)
    losses = jnp.stack(
        [jnp.sum(lp) for lp in (lp0, lp1, lp2, lp3)]) / (N * D)
    return xq, indices, losses


# speedup vs baseline: 1.0268x; 1.0268x over previous
import functools
import jax
import jax.numpy as jnp
from jax import lax
from jax.experimental import pallas as pl
from jax.experimental.pallas import tpu as pltpu
from jax.experimental.pallas import tpu_sc as plsc

_T = 512
_TF = 4096

_NC = 2
_NS = 16
_NW = _NC * _NS


def _argmin_dist(r, cbm2, cb2row, qoff):
    T = r.shape[0]
    K = cbm2.shape[0]
    mmn = lax.dot_general(
        r.astype(jnp.bfloat16), cbm2,
        (((1,), (1,)), ((), ())),
        preferred_element_type=jnp.float32,
    )
    r2 = jnp.sum(r * r, axis=1, keepdims=True)
    dist = (r2 + mmn) + cb2row
    m = jnp.min(dist, axis=1, keepdims=True)
    iota = lax.broadcasted_iota(jnp.int32, (1, K), 1).astype(jnp.float32)
    idxf = jnp.min(jnp.where(dist == m, iota, float(K)), axis=1)
    return (idxf + float(qoff)).astype(jnp.int32)


def _make_body_first(qoff):
    def body(x_ref, cbm2_ref, cb2_ref, idxg_ref):
        idxg_ref[0, :] = _argmin_dist(
            x_ref[...], cbm2_ref[...], cb2_ref[...], qoff)
    return body


def _make_body_mid(qoff):
    def body(rprev_ref, quant_ref, cbm2_ref, cb2_ref, idxg_ref, r_ref, lp_ref):
        r = rprev_ref[...] - quant_ref[...]
        r_ref[...] = r
        lp_ref[0, 0, :] = jnp.sum(r * r, axis=0)
        idxg_ref[0, :] = _argmin_dist(r, cbm2_ref[...], cb2_ref[...], qoff)
    return body


def _body_final(x_ref, rprev_ref, quant_ref, xq_ref, lp_ref):
    r = rprev_ref[...] - quant_ref[...]
    lp_ref[0, 0, :] = jnp.sum(r * r, axis=0)
    xq_ref[...] = x_ref[...] - r


def _body_cb2(cb_ref, out_ref):
    cb = cb_ref[...]
    out_ref[...] = jnp.sum(cb * cb, axis=2)


def _cb2_all(codebooks):
    Q, K, D = codebooks.shape
    return pl.pallas_call(
        _body_cb2,
        in_specs=[pl.BlockSpec((Q, K, D), lambda: (0, 0, 0))],
        out_specs=pl.BlockSpec((Q, K), lambda: (0, 0)),
        out_shape=jax.ShapeDtypeStruct((Q, K), jnp.float32),
    )(codebooks)


def _tc_first(x, cbm2, cb2, q):
    N, D = x.shape
    K = cbm2.shape[0]
    G = N // _T
    return pl.pallas_call(
        _make_body_first(q * K),
        grid=(G,),
        in_specs=[
            pl.BlockSpec((_T, D), lambda i: (i, 0)),
            pl.BlockSpec((K, D), lambda i: (0, 0)),
            pl.BlockSpec((1, K), lambda i: (0, 0)),
        ],
        out_specs=pl.BlockSpec((1, _T), lambda i: (0, i)),
        out_shape=jax.ShapeDtypeStruct((1, N), jnp.int32),
    )(x, cbm2, cb2)


def _tc_mid(rprev, quant, cbm2, cb2, q):
    N, D = rprev.shape
    K = cbm2.shape[0]
    G = N // _T
    return pl.pallas_call(
        _make_body_mid(q * K),
        grid=(G,),
        in_specs=[
            pl.BlockSpec((_T, D), lambda i: (i, 0)),
            pl.BlockSpec((_T, D), lambda i: (i, 0)),
            pl.BlockSpec((K, D), lambda i: (0, 0)),
            pl.BlockSpec((1, K), lambda i: (0, 0)),
        ],
        out_specs=[
            pl.BlockSpec((1, _T), lambda i: (0, i)),
            pl.BlockSpec((_T, D), lambda i: (i, 0)),
            pl.BlockSpec((1, 1, D), lambda i: (i, 0, 0)),
        ],
        out_shape=[
            jax.ShapeDtypeStruct((1, N), jnp.int32),
            jax.ShapeDtypeStruct((N, D), jnp.float32),
            jax.ShapeDtypeStruct((G, 1, D), jnp.float32),
        ],
    )(rprev, quant, cbm2, cb2)


def _tc_final(x, rprev, quant):
    N, D = x.shape
    G = N // _TF
    return pl.pallas_call(
        _body_final,
        grid=(G,),
        in_specs=[
            pl.BlockSpec((_TF, D), lambda i: (i, 0)),
            pl.BlockSpec((_TF, D), lambda i: (i, 0)),
            pl.BlockSpec((_TF, D), lambda i: (i, 0)),
        ],
        out_specs=[
            pl.BlockSpec((_TF, D), lambda i: (i, 0)),
            pl.BlockSpec((1, 1, D), lambda i: (i, 0, 0)),
        ],
        out_shape=[
            jax.ShapeDtypeStruct((N, D), jnp.float32),
            jax.ShapeDtypeStruct((G, 1, D), jnp.float32),
        ],
    )(x, rprev, quant)


def _sc_gather(flat_cb, idxg):
    KQ, D = flat_cb.shape
    N = idxg.shape[1]
    bpw = N // _NW
    nchunk = bpw // 128
    mesh = plsc.VectorSubcoreMesh(core_axis_name="c", subcore_axis_name="s")

    @functools.partial(
        pl.kernel, mesh=mesh,
        compiler_params=pltpu.CompilerParams(use_tc_tiling_on_sc=False),
        out_type=jax.ShapeDtypeStruct((N, D), jnp.float32),
        scratch_types=[
            pltpu.VMEM((bpw,), jnp.int32),
            pltpu.VMEM((bpw, D), jnp.float32),
            pltpu.SemaphoreType.DMA,
        ],
    )
    def k(cb_hbm, idx_hbm, out_hbm, idx_v, rows_v, sem):
        wid = lax.axis_index("s") * _NC + lax.axis_index("c")
        pltpu.sync_copy(idx_hbm.at[0, pl.ds(wid * bpw, bpw)], idx_v)
        copies = [
            pltpu.async_copy(
                cb_hbm.at[idx_v.at[pl.ds(j * 128, 128)]],
                rows_v.at[pl.ds(j * 128, 128)], sem)
            for j in range(nchunk)
        ]
        for c in copies:
            c.wait()
        pltpu.sync_copy(rows_v, out_hbm.at[pl.ds(wid * bpw, bpw)])

    return k(flat_cb, idxg)


def kernel(x, codebooks):
    B, S, D = x.shape
    Q, K, _ = codebooks.shape
    N = B * S
    xf = x.reshape(N, D)
    flat_cb = codebooks.reshape(Q * K, D)

    cbm2 = (-2.0 * codebooks).astype(jnp.bfloat16)
    cb2 = _cb2_all(codebooks)

    ig0 = _tc_first(xf, cbm2[0], cb2[0:1], 0)
    q0 = _sc_gather(flat_cb, ig0)
    ig1, r1, lp0 = _tc_mid(xf, q0, cbm2[1], cb2[1:2], 1)
    q1 = _sc_gather(flat_cb, ig1)
    ig2, r2, lp1 = _tc_mid(r1, q1, cbm2[2], cb2[2:3], 2)
    q2 = _sc_gather(flat_cb, ig2)
    ig3, r3, lp2 = _tc_mid(r2, q2, cbm2[3], cb2[3:4], 3)
    q3 = _sc_gather(flat_cb, ig3)
    xq_f, lp3 = _tc_final(xf, r3, q3)

    xq = xq_f.reshape(B, S, D)
    indices = jnp.stack(
        [ig0[0], ig1[0] - K, ig2[0] - 2 * K, ig3[0] - 3 * K],
        axis=-1).reshape(B, S, Q)
    losses = jnp.stack(
        [jnp.sum(lp) for lp in (lp0, lp1, lp2, lp3)]) / (N * D)
    return xq, indices, losses


# speedup vs baseline: 1.0378x; 1.0108x over previous
import functools
import jax
import jax.numpy as jnp
from jax import lax
from jax.experimental import pallas as pl
from jax.experimental.pallas import tpu as pltpu
from jax.experimental.pallas import tpu_sc as plsc

_T = 512
_TF = 4096

_NC = 2
_NS = 16
_NW = _NC * _NS


def _argmin_dist(r, cbm2, cb2row, qoff):
    T = r.shape[0]
    K = cbm2.shape[0]
    mmn = lax.dot_general(
        r.astype(jnp.bfloat16), cbm2,
        (((1,), (1,)), ((), ())),
        preferred_element_type=jnp.float32,
    )
    r2 = jnp.sum(r * r, axis=1, keepdims=True)
    dist = (r2 + mmn) + cb2row
    m = jnp.min(dist, axis=1, keepdims=True)
    iota = lax.broadcasted_iota(jnp.int32, (1, K), 1).astype(jnp.float32)
    idxf = jnp.min(jnp.where(dist == m, iota, float(K)), axis=1)
    return (idxf + float(qoff)).astype(jnp.int32)


def _make_body_first(qoff):
    def body(x_ref, cbm2_ref, cb2_ref, idxg_ref):
        idxg_ref[...] = _argmin_dist(
            x_ref[...], cbm2_ref[0], cb2_ref[0], qoff)
    return body


def _make_body_mid(qoff):
    def body(rprev_ref, quant_ref, cbm2_ref, cb2_ref, idxg_ref, r_ref, lp_ref):
        r = rprev_ref[...] - quant_ref[...]
        r_ref[...] = r
        lp_ref[0, 0, :] = jnp.sum(r * r, axis=0)
        idxg_ref[...] = _argmin_dist(r, cbm2_ref[0], cb2_ref[0], qoff)
    return body


def _body_final(x_ref, rprev_ref, quant_ref, xq_ref, lp_ref):
    r = rprev_ref[...] - quant_ref[...]
    lp_ref[0, 0, :] = jnp.sum(r * r, axis=0)
    xq_ref[...] = x_ref[...] - r


def _body_cb2(cb_ref, cbm2_ref, cb2_ref):
    cb = cb_ref[...]
    cbm2_ref[...] = (-2.0 * cb).astype(jnp.bfloat16)
    cb2_ref[...] = jnp.sum(cb * cb, axis=2)[:, None, :]


def _cb2_all(codebooks):
    Q, K, D = codebooks.shape
    return pl.pallas_call(
        _body_cb2,
        in_specs=[pl.BlockSpec((Q, K, D), lambda: (0, 0, 0))],
        out_specs=[
            pl.BlockSpec((Q, K, D), lambda: (0, 0, 0)),
            pl.BlockSpec((Q, 1, K), lambda: (0, 0, 0)),
        ],
        out_shape=[
            jax.ShapeDtypeStruct((Q, K, D), jnp.bfloat16),
            jax.ShapeDtypeStruct((Q, 1, K), jnp.float32),
        ],
    )(codebooks)


def _tc_first(x, cbm2, cb2, q):
    N, D = x.shape
    Q, K, _ = cbm2.shape
    G = N // _T
    return pl.pallas_call(
        _make_body_first(q * K),
        grid=(G,),
        in_specs=[
            pl.BlockSpec((_T, D), lambda i: (i, 0)),
            pl.BlockSpec((1, K, D), lambda i: (q, 0, 0)),
            pl.BlockSpec((1, 1, K), lambda i: (q, 0, 0)),
        ],
        out_specs=pl.BlockSpec((_T,), lambda i: (i,)),
        out_shape=jax.ShapeDtypeStruct((N,), jnp.int32),
    )(x, cbm2, cb2)


def _tc_mid(rprev, quant, cbm2, cb2, q):
    N, D = rprev.shape
    Q, K, _ = cbm2.shape
    G = N // _T
    return pl.pallas_call(
        _make_body_mid(q * K),
        grid=(G,),
        in_specs=[
            pl.BlockSpec((_T, D), lambda i: (i, 0)),
            pl.BlockSpec((_T, D), lambda i: (i, 0)),
            pl.BlockSpec((1, K, D), lambda i: (q, 0, 0)),
            pl.BlockSpec((1, 1, K), lambda i: (q, 0, 0)),
        ],
        out_specs=[
            pl.BlockSpec((_T,), lambda i: (i,)),
            pl.BlockSpec((_T, D), lambda i: (i, 0)),
            pl.BlockSpec((1, 1, D), lambda i: (i, 0, 0)),
        ],
        out_shape=[
            jax.ShapeDtypeStruct((N,), jnp.int32),
            jax.ShapeDtypeStruct((N, D), jnp.float32),
            jax.ShapeDtypeStruct((G, 1, D), jnp.float32),
        ],
    )(rprev, quant, cbm2, cb2)


def _tc_final(x, rprev, quant):
    N, D = x.shape
    G = N // _TF
    return pl.pallas_call(
        _body_final,
        grid=(G,),
        in_specs=[
            pl.BlockSpec((_TF, D), lambda i: (i, 0)),
            pl.BlockSpec((_TF, D), lambda i: (i, 0)),
            pl.BlockSpec((_TF, D), lambda i: (i, 0)),
        ],
        out_specs=[
            pl.BlockSpec((_TF, D), lambda i: (i, 0)),
            pl.BlockSpec((1, 1, D), lambda i: (i, 0, 0)),
        ],
        out_shape=[
            jax.ShapeDtypeStruct((N, D), jnp.float32),
            jax.ShapeDtypeStruct((G, 1, D), jnp.float32),
        ],
    )(x, rprev, quant)


def _sc_gather(flat_cb, idxg):
    KQ, D = flat_cb.shape
    N = idxg.shape[0]
    bpw = N // _NW
    nchunk = bpw // 128
    mesh = plsc.VectorSubcoreMesh(core_axis_name="c", subcore_axis_name="s")

    @functools.partial(
        pl.kernel, mesh=mesh,
        compiler_params=pltpu.CompilerParams(use_tc_tiling_on_sc=False),
        out_type=jax.ShapeDtypeStruct((N, D), jnp.float32),
        scratch_types=[
            pltpu.VMEM((bpw,), jnp.int32),
            pltpu.VMEM((bpw, D), jnp.float32),
            pltpu.SemaphoreType.DMA,
        ],
    )
    def k(cb_hbm, idx_hbm, out_hbm, idx_v, rows_v, sem):
        wid = lax.axis_index("s") * _NC + lax.axis_index("c")
        pltpu.sync_copy(idx_hbm.at[pl.ds(wid * bpw, bpw)], idx_v)
        copies = [
            pltpu.async_copy(
                cb_hbm.at[idx_v.at[pl.ds(j * 128, 128)]],
                rows_v.at[pl.ds(j * 128, 128)], sem)
            for j in range(nchunk)
        ]
        for c in copies:
            c.wait()
        pltpu.sync_copy(rows_v, out_hbm.at[pl.ds(wid * bpw, bpw)])

    return k(flat_cb, idxg)


def kernel(x, codebooks):
    B, S, D = x.shape
    Q, K, _ = codebooks.shape
    N = B * S
    xf = x.reshape(N, D)
    flat_cb = codebooks.reshape(Q * K, D)

    cbm2, cb2 = _cb2_all(codebooks)

    ig0 = _tc_first(xf, cbm2, cb2, 0)
    q0 = _sc_gather(flat_cb, ig0)
    ig1, r1, lp0 = _tc_mid(xf, q0, cbm2, cb2, 1)
    q1 = _sc_gather(flat_cb, ig1)
    ig2, r2, lp1 = _tc_mid(r1, q1, cbm2, cb2, 2)
    q2 = _sc_gather(flat_cb, ig2)
    ig3, r3, lp2 = _tc_mid(r2, q2, cbm2, cb2, 3)
    q3 = _sc_gather(flat_cb, ig3)
    xq_f, lp3 = _tc_final(xf, r3, q3)

    xq = xq_f.reshape(B, S, D)
    indices = jnp.stack(
        [ig0, ig1 - K, ig2 - 2 * K, ig3 - 3 * K],
        axis=-1).reshape(B, S, Q)
    losses = jnp.stack(
        [jnp.sum(lp) for lp in (lp0, lp1, lp2, lp3)]) / (N * D)
    return xq, indices, losses


# speedup vs baseline: 1.0651x; 1.0262x over previous
import functools
import jax
import jax.numpy as jnp
from jax import lax
from jax.experimental import pallas as pl
from jax.experimental.pallas import tpu as pltpu
from jax.experimental.pallas import tpu_sc as plsc

_T = 512
_TF = 4096

_NC = 2
_NS = 16
_NW = _NC * _NS


def _argmin_dist(r, cbm2, cb2row, qoff):
    T = r.shape[0]
    K = cbm2.shape[0]
    mmn = lax.dot_general(
        r.astype(jnp.bfloat16), cbm2,
        (((1,), (1,)), ((), ())),
        preferred_element_type=jnp.float32,
    )
    r2 = jnp.sum(r * r, axis=1, keepdims=True)
    dist = (r2 + mmn) + cb2row
    m = jnp.min(dist, axis=1, keepdims=True)
    iota = lax.broadcasted_iota(jnp.int32, (1, K), 1).astype(jnp.float32)
    idxf = jnp.min(jnp.where(dist == m, iota, float(K)), axis=1)
    return (idxf + float(qoff)).astype(jnp.int32)


def _store_idx_rows(idxg_ref, ig):
    rows = ig.reshape(_T // 128, 128)
    odd = pl.program_id(0) % 2

    @pl.when(odd == 0)
    def _():
        idxg_ref[0:4, :] = rows

    @pl.when(odd == 1)
    def _():
        idxg_ref[4:8, :] = rows


def _make_body_first(qoff):
    def body(x_ref, cbm2_ref, cb2_ref, idxg_ref):
        _store_idx_rows(idxg_ref, _argmin_dist(
            x_ref[...], cbm2_ref[0], cb2_ref[0], qoff))
    return body


def _make_body_mid(qoff):
    def body(rprev_ref, quant_ref, cbm2_ref, cb2_ref, idxg_ref, r_ref, lp_ref):
        r = rprev_ref[...] - quant_ref[...]
        r_ref[...] = r
        lp_ref[0, 0, :] = jnp.sum(r * r, axis=0)
        _store_idx_rows(idxg_ref, _argmin_dist(r, cbm2_ref[0], cb2_ref[0], qoff))
    return body


def _body_final(x_ref, rprev_ref, quant_ref, xq_ref, lp_ref):
    r = rprev_ref[...] - quant_ref[...]
    lp_ref[0, 0, :] = jnp.sum(r * r, axis=0)
    xq_ref[...] = x_ref[...] - r


def _body_cb2(cb_ref, cbm2_ref, cb2_ref):
    cb = cb_ref[...]
    cbm2_ref[...] = (-2.0 * cb).astype(jnp.bfloat16)
    cb2_ref[...] = jnp.sum(cb * cb, axis=2)[:, None, :]


def _cb2_all(codebooks):
    Q, K, D = codebooks.shape
    return pl.pallas_call(
        _body_cb2,
        in_specs=[pl.BlockSpec((Q, K, D), lambda: (0, 0, 0))],
        out_specs=[
            pl.BlockSpec((Q, K, D), lambda: (0, 0, 0)),
            pl.BlockSpec((Q, 1, K), lambda: (0, 0, 0)),
        ],
        out_shape=[
            jax.ShapeDtypeStruct((Q, K, D), jnp.bfloat16),
            jax.ShapeDtypeStruct((Q, 1, K), jnp.float32),
        ],
    )(codebooks)


def _tc_first(x, cbm2, cb2, q):
    N, D = x.shape
    Q, K, _ = cbm2.shape
    G = N // _T
    return pl.pallas_call(
        _make_body_first(q * K),
        grid=(G,),
        in_specs=[
            pl.BlockSpec((_T, D), lambda i: (i, 0)),
            pl.BlockSpec((1, K, D), lambda i: (q, 0, 0)),
            pl.BlockSpec((1, 1, K), lambda i: (q, 0, 0)),
        ],
        out_specs=pl.BlockSpec((8, 128), lambda i: (i // 2, 0)),
        out_shape=jax.ShapeDtypeStruct((N // 128, 128), jnp.int32),
    )(x, cbm2, cb2)


def _tc_mid(rprev, quant, cbm2, cb2, q):
    N, D = rprev.shape
    Q, K, _ = cbm2.shape
    G = N // _T
    return pl.pallas_call(
        _make_body_mid(q * K),
        grid=(G,),
        in_specs=[
            pl.BlockSpec((_T, D), lambda i: (i, 0)),
            pl.BlockSpec((_T, D), lambda i: (i, 0)),
            pl.BlockSpec((1, K, D), lambda i: (q, 0, 0)),
            pl.BlockSpec((1, 1, K), lambda i: (q, 0, 0)),
        ],
        out_specs=[
            pl.BlockSpec((8, 128), lambda i: (i // 2, 0)),
            pl.BlockSpec((_T, D), lambda i: (i, 0)),
            pl.BlockSpec((1, 1, D), lambda i: (i, 0, 0)),
        ],
        out_shape=[
            jax.ShapeDtypeStruct((N // 128, 128), jnp.int32),
            jax.ShapeDtypeStruct((N, D), jnp.float32),
            jax.ShapeDtypeStruct((G, 1, D), jnp.float32),
        ],
    )(rprev, quant, cbm2, cb2)


def _tc_final(x, rprev, quant):
    N, D = x.shape
    G = N // _TF
    return pl.pallas_call(
        _body_final,
        grid=(G,),
        in_specs=[
            pl.BlockSpec((_TF, D), lambda i: (i, 0)),
            pl.BlockSpec((_TF, D), lambda i: (i, 0)),
            pl.BlockSpec((_TF, D), lambda i: (i, 0)),
        ],
        out_specs=[
            pl.BlockSpec((_TF, D), lambda i: (i, 0)),
            pl.BlockSpec((1, 1, D), lambda i: (i, 0, 0)),
        ],
        out_shape=[
            jax.ShapeDtypeStruct((N, D), jnp.float32),
            jax.ShapeDtypeStruct((G, 1, D), jnp.float32),
        ],
    )(x, rprev, quant)


def _sc_gather(flat_cb, idxg):
    KQ, D = flat_cb.shape
    N = idxg.shape[0] * 128
    bpw = N // _NW
    nchunk = bpw // 128
    mesh = plsc.VectorSubcoreMesh(core_axis_name="c", subcore_axis_name="s")

    @functools.partial(
        pl.kernel, mesh=mesh,
        compiler_params=pltpu.CompilerParams(use_tc_tiling_on_sc=False),
        out_type=jax.ShapeDtypeStruct((N, D), jnp.float32),
        scratch_types=[
            pltpu.VMEM((nchunk, 128), jnp.int32),
            pltpu.VMEM((bpw, D), jnp.float32),
            pltpu.SemaphoreType.DMA,
        ],
    )
    def k(cb_hbm, idx_hbm, out_hbm, idx_v, rows_v, sem):
        wid = lax.axis_index("s") * _NC + lax.axis_index("c")
        pltpu.sync_copy(idx_hbm.at[pl.ds(wid * nchunk, nchunk)], idx_v)
        copies = [
            pltpu.async_copy(
                cb_hbm.at[idx_v.at[j]],
                rows_v.at[pl.ds(j * 128, 128)], sem)
            for j in range(nchunk)
        ]
        for c in copies:
            c.wait()
        pltpu.sync_copy(rows_v, out_hbm.at[pl.ds(wid * bpw, bpw)])

    return k(flat_cb, idxg)


def kernel(x, codebooks):
    B, S, D = x.shape
    Q, K, _ = codebooks.shape
    N = B * S
    xf = x.reshape(N, D)
    flat_cb = codebooks.reshape(Q * K, D)

    cbm2, cb2 = _cb2_all(codebooks)

    ig0 = _tc_first(xf, cbm2, cb2, 0)
    q0 = _sc_gather(flat_cb, ig0)
    ig1, r1, lp0 = _tc_mid(xf, q0, cbm2, cb2, 1)
    q1 = _sc_gather(flat_cb, ig1)
    ig2, r2, lp1 = _tc_mid(r1, q1, cbm2, cb2, 2)
    q2 = _sc_gather(flat_cb, ig2)
    ig3, r3, lp2 = _tc_mid(r2, q2, cbm2, cb2, 3)
    q3 = _sc_gather(flat_cb, ig3)
    xq_f, lp3 = _tc_final(xf, r3, q3)

    xq = xq_f.reshape(B, S, D)
    indices = jnp.stack(
        [ig0.reshape(N), ig1.reshape(N) - K,
         ig2.reshape(N) - 2 * K, ig3.reshape(N) - 3 * K],
        axis=-1).reshape(B, S, Q)
    losses = jnp.stack(
        [jnp.sum(lp) for lp in (lp0, lp1, lp2, lp3)]) / (N * D)
    return xq, indices, losses
